# Initial kernel scaffold; baseline (speedup 1.0000x reference)
import jax
import jax.numpy as jnp
from jax.experimental import pallas as pl


def kernel(x, edge_index, edge_attr, W_red, b_red, W1, b1, g1, be1, W2, b2, g2, be2, W_out, b_out):
    raise NotImplementedError("write your pallas kernel here")


# speedup vs baseline: 7.4241x; 7.4241x over previous
import functools
import jax
import jax.numpy as jnp
from jax import lax
from jax.experimental import pallas as pl
from jax.experimental.pallas import tpu as pltpu
from jax.experimental.pallas import tpu_sc as plsc

F32 = jnp.float32

SELU_ALPHA = 1.6732632423543772848170429916717
SELU_SCALE = 1.0507009873554804934193349852946
BN_EPS = 1e-5

CH = 80
HH = 64


def _selu(v):
    return SELU_SCALE * jnp.where(v > 0, v, SELU_ALPHA * (jnp.exp(v) - 1.0))


def _tc1_body(x_ref, w_ref, b_ref, o_ref):
    xw = jnp.dot(x_ref[...], w_ref[...], preferred_element_type=F32)
    o_ref[...] = _selu(xw + b_ref[...])


def _dinv_col(degt_ref, n):
    deg = degt_ref[...][:n, 0:1] + degt_ref[...][:n, 1:2] + 1.0
    return lax.rsqrt(deg)


def _tc2_body(n, n_pad, h_ref, w_ref, degt_ref, y_ref):
    dinv = _dinv_col(degt_ref, n)
    xw = jnp.dot(h_ref[...], w_ref[...], preferred_element_type=F32)
    y = xw * dinv
    y_ref[0:n, :] = y[:, 0:HH]
    y_ref[n_pad:n_pad + n, :] = y[:, HH:2 * HH]


def _post_conv(acc_ref, y_ref, dinv, b_ref, g_ref, be_ref, id_ref, n, n_pad):
    a = jnp.concatenate(
        [acc_ref[0, 0:n, :] + y_ref[0:n, :],
         acc_ref[1, 0:n, :] + y_ref[n_pad:n_pad + n, :]], axis=1)
    conv = a * dinv + b_ref[...]
    bn = conv * (1.0 / (1.0 + BN_EPS) ** 0.5) * g_ref[...] + be_ref[...]
    return _selu(bn) + id_ref[...]


def _tc3_body(n, n_pad, acc_ref, y_ref, degt_ref, id_ref, w2_ref,
              b1_ref, g1_ref, be1_ref, h2_ref, y2_ref):
    dinv = _dinv_col(degt_ref, n)
    h2 = _post_conv(acc_ref, y_ref, dinv, b1_ref, g1_ref, be1_ref,
                    id_ref, n, n_pad)
    h2_ref[...] = h2
    y2 = jnp.dot(h2, w2_ref[...], preferred_element_type=F32) * dinv
    y2_ref[0:n, :] = y2[:, 0:HH]
    y2_ref[n_pad:n_pad + n, :] = y2[:, HH:2 * HH]


def _tc4_body(n, n_pad, acc_ref, y_ref, degt_ref, id_ref, wo_ref,
              b2_ref, g2_ref, be2_ref, bo_ref, o_ref):
    dinv = _dinv_col(degt_ref, n)
    h3 = _post_conv(acc_ref, y_ref, dinv, b2_ref, g2_ref, be2_ref,
                    id_ref, n, n_pad)
    o_ref[...] = jnp.dot(h3, wo_ref[...], preferred_element_type=F32) + bo_ref[...]


def _sc_mesh():
    return plsc.VectorSubcoreMesh(core_axis_name="c", subcore_axis_name="s")


def _make_deg_kernel(n_pad, nch):
    stripe = n_pad // 16

    @functools.partial(
        pl.kernel,
        out_type=jax.ShapeDtypeStruct((2 * n_pad,), F32),
        mesh=_sc_mesh(),
        scratch_types=[
            pltpu.VMEM((nch, CH), jnp.int32),
            pltpu.VMEM((nch, CH), F32),
            pltpu.VMEM((stripe,), F32),
            pltpu.VMEM_SHARED((n_pad,), F32),
        ],
    )
    def deg_kernel(col_hbm, w_hbm, out_hbm, col_v, w_v, buf_v, deg_sp):
        c = lax.axis_index("c")
        s = lax.axis_index("s")
        wid = c * 16 + s
        pltpu.sync_copy(col_hbm.at[wid], col_v)
        pltpu.sync_copy(w_hbm.at[wid], w_v)

        def zero(j, carry):
            buf_v[pl.ds(j * 16, 16)] = jnp.zeros((16,), F32)
            return carry

        lax.fori_loop(0, stripe // 16, zero, 0)
        pltpu.sync_copy(buf_v, deg_sp.at[pl.ds(s * stripe, stripe)])
        plsc.subcore_barrier()

        def chunk(ci, carry):
            pltpu.sync_copy(w_v.at[ci], deg_sp.at[col_v.at[ci]], add=True)
            return carry

        lax.fori_loop(0, nch, chunk, 0)
        plsc.subcore_barrier()
        pltpu.sync_copy(deg_sp.at[pl.ds(s * stripe, stripe)], buf_v)
        pltpu.sync_copy(buf_v, out_hbm.at[pl.ds(c * n_pad + s * stripe, stripe)])

    return deg_kernel


def _make_mp_kernel(n_pad, nch):
    stripe = n_pad // 16

    @functools.partial(
        pl.kernel,
        out_type=jax.ShapeDtypeStruct((2, n_pad, HH), F32),
        mesh=_sc_mesh(),
        scratch_types=[
            pltpu.VMEM((nch, CH), jnp.int32),
            pltpu.VMEM((nch, CH), jnp.int32),
            pltpu.VMEM((nch, CH), F32),
            pltpu.VMEM((CH, HH), F32),
            pltpu.SemaphoreType.DMA,
            pltpu.VMEM_SHARED((n_pad, HH), F32),
        ],
        compiler_params=pltpu.CompilerParams(use_tc_tiling_on_sc=False),
    )
    def mp_kernel(y_hbm, row_hbm, col_hbm, w_hbm, out_hbm,
                  row_v, col_v, w_v, rows_v, sem, acc_sp):
        c = lax.axis_index("c")
        s = lax.axis_index("s")
        wid = c * 16 + s
        pltpu.sync_copy(row_hbm.at[wid], row_v)
        pltpu.sync_copy(col_hbm.at[s], col_v)
        pltpu.sync_copy(w_hbm.at[s], w_v)

        def zero(j, carry):
            for q in range(HH // 16):
                rows_v[j, pl.ds(q * 16, 16)] = jnp.zeros((16,), F32)
            return carry

        lax.fori_loop(0, CH, zero, 0)
        for b in range(stripe // CH):
            pltpu.sync_copy(rows_v, acc_sp.at[pl.ds(s * stripe + b * CH, CH)])
        plsc.subcore_barrier()

        def chunk(ci, carry):
            pltpu.async_copy(y_hbm.at[row_v.at[ci]], rows_v, sem).wait()

            def group(g, gcarry):
                wg = w_v[ci, pl.ds(g * 16, 16)]
                for t in range(16):
                    wj = jnp.broadcast_to(wg[t], (16,))
                    j = g * 16 + t
                    for q in range(HH // 16):
                        sl = pl.ds(q * 16, 16)
                        rows_v[j, sl] = rows_v[j, sl] * wj
                return gcarry

            lax.fori_loop(0, CH // 16, group, 0)
            pltpu.sync_copy(rows_v, acc_sp.at[col_v.at[ci]], add=True)
            return carry

        lax.fori_loop(0, nch, chunk, 0)
        plsc.subcore_barrier()
        for b in range(stripe // CH):
            pltpu.sync_copy(acc_sp.at[pl.ds(s * stripe + b * CH, CH)], rows_v)
            pltpu.sync_copy(rows_v,
                            out_hbm.at[c, pl.ds(s * stripe + b * CH, CH)])

    return mp_kernel


def kernel(x, edge_index, edge_attr, W_red, b_red, W1, b1, g1, be1,
           W2, b2, g2, be2, W_out, b_out):
    n, f_in = x.shape
    h_dim = W_red.shape[1]
    c_dim = W_out.shape[1]
    e = edge_index.shape[1]

    n_pad = ((n + 255) // 256) * 256
    nch_d = e // (32 * CH)
    nch_m = e // (16 * CH)

    row = edge_index[0]
    col = edge_index[1]
    col_d = col.reshape(32, nch_d, CH)
    w_d = edge_attr.reshape(32, nch_d, CH)
    row_m = jnp.concatenate([row, row + n_pad]).reshape(32, nch_m, CH)
    col_m = col.reshape(16, nch_m, CH)
    w_m = edge_attr.reshape(16, nch_m, CH)

    deg_kernel = _make_deg_kernel(n_pad, nch_d)
    mp_kernel = _make_mp_kernel(n_pad, nch_m)

    degp = deg_kernel(col_d, w_d)
    degt = jnp.transpose(degp.reshape(2, n_pad))

    h = pl.pallas_call(
        _tc1_body,
        out_shape=jax.ShapeDtypeStruct((n, h_dim), F32),
    )(x, W_red, b_red.reshape(1, h_dim))

    y1 = pl.pallas_call(
        functools.partial(_tc2_body, n, n_pad),
        out_shape=jax.ShapeDtypeStruct((2 * n_pad, HH), F32),
    )(h, W1, degt)

    acc1 = mp_kernel(y1, row_m, col_m, w_m)

    h2, y2 = pl.pallas_call(
        functools.partial(_tc3_body, n, n_pad),
        out_shape=[
            jax.ShapeDtypeStruct((n, h_dim), F32),
            jax.ShapeDtypeStruct((2 * n_pad, HH), F32),
        ],
    )(acc1, y1, degt, h, W2, b1.reshape(1, h_dim), g1.reshape(1, h_dim),
      be1.reshape(1, h_dim))

    acc2 = mp_kernel(y2, row_m, col_m, w_m)

    out = pl.pallas_call(
        functools.partial(_tc4_body, n, n_pad),
        out_shape=jax.ShapeDtypeStruct((n, c_dim), F32),
    )(acc2, y2, degt, h2, W_out, b2.reshape(1, h_dim), g2.reshape(1, h_dim),
      be2.reshape(1, h_dim), b_out.reshape(1, c_dim))

    return out
